# baseline (speedup 1.0000x reference)
import functools

import jax
import jax.numpy as jnp
from jax import lax
from jax.experimental import pallas as pl
from jax.experimental.pallas import tpu as pltpu
from jax.experimental.pallas import tpu_sc as plsc


_STREAM = 128
_CHUNK = 1024
_PACK_BLK = 4096
_NSLAB = 5


def _pack_body(x_ref, w_ref, o_ref):
    x = x_ref[...]
    w = w_ref[...]
    blk = o_ref.shape[0]
    ya = lax.dot_general(
        x[:, :blk], w, (((0,), (1,)), ((), ())),
        preferred_element_type=jnp.float32,
    )
    yb = lax.dot_general(
        x[:, blk:], w, (((0,), (1,)), ((), ())),
        preferred_element_type=jnp.float32,
    )
    o_ref[...] = jnp.concatenate([ya, yb], axis=1)


def _pack_table(emb_table, align_w):
    v, d = emb_table.shape
    nblk = -(-v // (2 * _PACK_BLK))
    embT = emb_table.T
    return pl.pallas_call(
        _pack_body,
        grid=(nblk,),
        in_specs=[
            pl.BlockSpec((d, 2 * _PACK_BLK), lambda i: (0, i)),
            pl.BlockSpec((d, d), lambda i: (0, 0)),
        ],
        out_specs=pl.BlockSpec((_PACK_BLK, 2 * d), lambda i: (i, 0)),
        out_shape=jax.ShapeDtypeStruct((nblk * _PACK_BLK, 2 * d), jnp.float32),
    )(embT, align_w)


@functools.lru_cache(maxsize=None)
def _sc_gather_fn(b, s, d, nc, ns):
    rows = b * s
    nw = nc * ns
    n_chunks = rows // _CHUNK // nw
    n_str = _CHUNK // _STREAM
    cps = b // _CHUNK
    hb = b // 2
    hc = hb // _CHUNK

    mesh = plsc.VectorSubcoreMesh(core_axis_name="c", subcore_axis_name="s")

    @functools.partial(
        pl.kernel,
        mesh=mesh,
        out_type=jax.ShapeDtypeStruct((rows // 2, 2 * d), jnp.float32),
        scratch_types=[
            pltpu.VMEM((_CHUNK,), jnp.int32),
            pltpu.VMEM((_CHUNK, d), jnp.float32),
            pltpu.SemaphoreType.DMA,
        ],
        compiler_params=pltpu.CompilerParams(use_tc_tiling_on_sc=False),
    )
    def gather_kernel(table_hbm, idx_hbm, out_hbm, idx_v, rows_v, sem):
        wid = lax.axis_index("s") * nc + lax.axis_index("c")

        def body(t, carry):
            c = wid * n_chunks + t
            si = c // cps
            q = c % cps
            pltpu.sync_copy(idx_hbm.at[si, pl.ds(q * _CHUNK, _CHUNK)], idx_v)
            copies = [
                pltpu.async_copy(
                    table_hbm.at[idx_v.at[pl.ds(j * _STREAM, _STREAM)]],
                    rows_v.at[pl.ds(j * _STREAM, _STREAM)],
                    sem,
                )
                for j in range(n_str)
            ]
            for cp in copies:
                cp.wait()
            half = q // hc
            p0 = si * hb + (q % hc) * _CHUNK
            pltpu.sync_copy(
                rows_v, out_hbm.at[pl.ds(p0, _CHUNK), pl.ds(half * d, d)]
            )
            return carry

        lax.fori_loop(0, n_chunks, body, 0)

    return gather_kernel


def _out_body(buf_ref, x_ref, ids_ref, pos_ref, o_ref, *, s0):
    x2 = x_ref[...]
    d = pos_ref.shape[0]
    eye = (
        lax.broadcasted_iota(jnp.int32, (d, d), 0)
        == lax.broadcasted_iota(jnp.int32, (d, d), 1)
    ).astype(jnp.float32)
    ya = lax.dot_general(
        eye, x2[:, :64], (((1,), (1,)), ((), ())),
        preferred_element_type=jnp.float32,
    )
    yb = lax.dot_general(
        eye, x2[:, 64:], (((1,), (1,)), ((), ())),
        preferred_element_type=jnp.float32,
    )
    y = jnp.concatenate([ya, yb], axis=1)
    y3 = lax.broadcast_in_dim(y, o_ref.shape, (1, 2))
    ids3 = lax.broadcast_in_dim(ids_ref[...], o_ref.shape, (0, 1, 2))
    n_pos = pos_ref.shape[1]
    onehot = (
        lax.broadcasted_iota(jnp.int32, (n_pos, 1), 0)
        == pl.program_id(0) + s0 + 1
    ).astype(jnp.float32)
    ps = lax.dot_general(
        pos_ref[...], onehot, (((1,), (0,)), ((), ())),
        preferred_element_type=jnp.float32,
    )
    pos3 = lax.broadcast_in_dim(ps, o_ref.shape, (1, 2))
    o_ref[...] = y3 + jnp.where(ids3 != 0, pos3, 0.0)


def _tc_out_slab(buf, gathered2, idsT3, posT, b, s, d, s0, ns_slab):
    n_pos = posT.shape[1]
    return pl.pallas_call(
        functools.partial(_out_body, s0=s0),
        grid=(ns_slab,),
        in_specs=[
            pl.BlockSpec(memory_space=pl.ANY),
            pl.BlockSpec((b // 2, 2 * d), lambda i: (i, 0)),
            pl.BlockSpec((1, 1, b), lambda i, s0=s0: (i + s0, 0, 0)),
            pl.BlockSpec((d, n_pos), lambda i: (0, 0)),
        ],
        out_specs=pl.BlockSpec((1, d, b), lambda i, s0=s0: (i + s0, 0, 0)),
        out_shape=jax.ShapeDtypeStruct((s, d, b), jnp.float32),
        input_output_aliases={0: 0},
    )(buf, gathered2, idsT3, posT)


def kernel(input, emb_table, align_w, pos_table):
    b, s = input.shape
    v, d = emb_table.shape

    tpack = _pack_table(emb_table, align_w)
    table_lin = tpack.reshape(tpack.shape[0] * 2, d)

    idsT = input.T.astype(jnp.int32)
    blki = idsT >> 13
    lk = idsT & 8191
    idsT_phys = (blki << 13) + ((lk & 4095) << 1) + (lk >> 12)

    info = plsc.get_sparse_core_info()
    nc, ns = info.num_cores, info.num_subcores

    ns_slab = s // _NSLAB
    gfn = _sc_gather_fn(b, ns_slab, d, nc, ns)
    slabs = [
        gfn(table_lin, lax.slice_in_dim(idsT_phys, k * ns_slab, (k + 1) * ns_slab))
        for k in range(_NSLAB)
    ]

    posT = pos_table.T
    idsT3 = idsT.reshape(s, 1, b)
    out = None
    for k in range(_NSLAB):
        if out is None:
            out = _tc_out_slab(
                jnp.zeros((s, d, b), jnp.float32), slabs[k], idsT3, posT,
                b, s, d, k * ns_slab, ns_slab,
            )
        else:
            out = _tc_out_slab(
                out, slabs[k], idsT3, posT, b, s, d, k * ns_slab, ns_slab
            )
    return lax.transpose(out, (2, 0, 1))


# speedup vs baseline: 1.1549x; 1.1549x over previous
import functools

import jax
import jax.numpy as jnp
from jax import lax
from jax.experimental import pallas as pl
from jax.experimental.pallas import tpu as pltpu
from jax.experimental.pallas import tpu_sc as plsc


_STREAM = 128
_CHUNK = 1024
_PACK_BLK = 4096
_NSLAB = 5


def _pack_body(x_ref, w_ref, o_ref):
    x = x_ref[...]
    w = w_ref[...]
    blk = o_ref.shape[0]
    ya = lax.dot_general(
        x[:, :blk], w, (((0,), (1,)), ((), ())),
        preferred_element_type=jnp.float32,
    )
    yb = lax.dot_general(
        x[:, blk:], w, (((0,), (1,)), ((), ())),
        preferred_element_type=jnp.float32,
    )
    o_ref[...] = jnp.concatenate([ya, yb], axis=1)


def _pack_table(emb_table, align_w):
    v, d = emb_table.shape
    nblk = -(-v // (2 * _PACK_BLK))
    embT = emb_table.T
    return pl.pallas_call(
        _pack_body,
        grid=(nblk,),
        in_specs=[
            pl.BlockSpec((d, 2 * _PACK_BLK), lambda i: (0, i)),
            pl.BlockSpec((d, d), lambda i: (0, 0)),
        ],
        out_specs=pl.BlockSpec((_PACK_BLK, 2 * d), lambda i: (i, 0)),
        out_shape=jax.ShapeDtypeStruct((nblk * _PACK_BLK, 2 * d), jnp.float32),
    )(embT, align_w)


@functools.lru_cache(maxsize=None)
def _sc_gather_fn(b, s, d, nc, ns):
    rows = b * s
    nw = nc * ns
    n_chunks = rows // _CHUNK // nw
    n_str = _CHUNK // _STREAM
    cps = b // _CHUNK
    hb = b // 2
    hc = hb // _CHUNK

    mesh = plsc.VectorSubcoreMesh(core_axis_name="c", subcore_axis_name="s")

    @functools.partial(
        pl.kernel,
        mesh=mesh,
        out_type=jax.ShapeDtypeStruct((rows // 2, 2 * d), jnp.float32),
        scratch_types=[
            pltpu.VMEM((_CHUNK,), jnp.int32),
            pltpu.VMEM((_CHUNK, d), jnp.float32),
            pltpu.SemaphoreType.DMA,
        ],
        compiler_params=pltpu.CompilerParams(use_tc_tiling_on_sc=False),
    )
    def gather_kernel(table_hbm, idx_hbm, out_hbm, idx_v, rows_v, sem):
        wid = lax.axis_index("s") * nc + lax.axis_index("c")

        def body(t, carry):
            c = wid * n_chunks + t
            si = c // cps
            q = c % cps
            pltpu.sync_copy(idx_hbm.at[si, pl.ds(q * _CHUNK, _CHUNK)], idx_v)
            copies = [
                pltpu.async_copy(
                    table_hbm.at[idx_v.at[pl.ds(j * _STREAM, _STREAM)]],
                    rows_v.at[pl.ds(j * _STREAM, _STREAM)],
                    sem,
                )
                for j in range(n_str)
            ]
            for cp in copies:
                cp.wait()
            half = q // hc
            p0 = si * hb + (q % hc) * _CHUNK
            pltpu.sync_copy(
                rows_v, out_hbm.at[pl.ds(p0, _CHUNK), pl.ds(half * d, d)]
            )
            return carry

        lax.fori_loop(0, n_chunks, body, 0)

    return gather_kernel


def _out_body(x_ref, ids_ref, pos_ref, o_ref, *, s0):
    x2 = x_ref[...]
    d = pos_ref.shape[0]
    eye = (
        lax.broadcasted_iota(jnp.int32, (d, d), 0)
        == lax.broadcasted_iota(jnp.int32, (d, d), 1)
    ).astype(jnp.float32)
    ya = lax.dot_general(
        eye, x2[:, :64], (((1,), (1,)), ((), ())),
        preferred_element_type=jnp.float32,
    )
    yb = lax.dot_general(
        eye, x2[:, 64:], (((1,), (1,)), ((), ())),
        preferred_element_type=jnp.float32,
    )
    y = jnp.concatenate([ya, yb], axis=1)
    y3 = lax.broadcast_in_dim(y, o_ref.shape, (1, 2))
    ids3 = lax.broadcast_in_dim(ids_ref[...], o_ref.shape, (0, 1, 2))
    n_pos = pos_ref.shape[1]
    onehot = (
        lax.broadcasted_iota(jnp.int32, (n_pos, 1), 0)
        == pl.program_id(0) + s0 + 1
    ).astype(jnp.float32)
    ps = lax.dot_general(
        pos_ref[...], onehot, (((1,), (0,)), ((), ())),
        preferred_element_type=jnp.float32,
    )
    pos3 = lax.broadcast_in_dim(ps, o_ref.shape, (1, 2))
    o_ref[...] = y3 + jnp.where(ids3 != 0, pos3, 0.0)


def _tc_out_slab(buf, gathered2, idsT3, posT, b, s, d, s0, ns_slab):
    n_pos = posT.shape[1]
    in_specs = [
        pl.BlockSpec((b // 2, 2 * d), lambda i: (i, 0)),
        pl.BlockSpec((1, 1, b), lambda i, s0=s0: (i + s0, 0, 0)),
        pl.BlockSpec((d, n_pos), lambda i: (0, 0)),
    ]
    args = (gathered2, idsT3, posT)
    aliases = {}
    if buf is not None:
        in_specs = [pl.BlockSpec(memory_space=pl.ANY)] + in_specs
        args = (buf,) + args
        aliases = {0: 0}
        body = functools.partial(_out_body_buf, s0=s0)
    else:
        body = functools.partial(_out_body, s0=s0)
    return pl.pallas_call(
        body,
        grid=(ns_slab,),
        in_specs=in_specs,
        out_specs=pl.BlockSpec((1, d, b), lambda i, s0=s0: (i + s0, 0, 0)),
        out_shape=jax.ShapeDtypeStruct((s, d, b), jnp.float32),
        input_output_aliases=aliases,
    )(*args)


def _out_body_buf(buf_ref, x_ref, ids_ref, pos_ref, o_ref, *, s0):
    _out_body(x_ref, ids_ref, pos_ref, o_ref, s0=s0)


def kernel(input, emb_table, align_w, pos_table):
    b, s = input.shape
    v, d = emb_table.shape

    tpack = _pack_table(emb_table, align_w)
    table_lin = tpack.reshape(tpack.shape[0] * 2, d)

    idsT = input.T.astype(jnp.int32)
    blki = idsT >> 13
    lk = idsT & 8191
    idsT_phys = (blki << 13) + ((lk & 4095) << 1) + (lk >> 12)

    info = plsc.get_sparse_core_info()
    nc, ns = info.num_cores, info.num_subcores

    ns_slab = s // _NSLAB
    gfn = _sc_gather_fn(b, ns_slab, d, nc, ns)
    slabs = [
        gfn(table_lin, lax.slice_in_dim(idsT_phys, k * ns_slab, (k + 1) * ns_slab))
        for k in range(_NSLAB)
    ]

    posT = pos_table.T
    idsT3 = idsT.reshape(s, 1, b)
    out = None
    for k in range(_NSLAB):
        out = _tc_out_slab(
            out, slabs[k], idsT3, posT, b, s, d, k * ns_slab, ns_slab
        )
    return lax.transpose(out, (2, 0, 1))
